# baseline (speedup 1.0000x reference)
import functools

import jax
import jax.numpy as jnp
from jax import lax
from jax.experimental import pallas as pl
from jax.experimental.pallas import tpu as pltpu
from jax.experimental.pallas import tpu_sc as plsc

_NUM_WORKERS = 32
_NBUF = 7


def _emb_kernel(n_chunks, per_w,
                tok_hbm, table_hbm, out_hbm, idx_v, rows_v, gsem, wsem):
    wid = lax.axis_index("s") * 2 + lax.axis_index("c")
    b0 = wid * per_w
    pltpu.sync_copy(tok_hbm.at[:, pl.ds(b0, per_w)], idx_v)

    def gather(j, p):
        return pltpu.make_async_copy(
            table_hbm.at[idx_v.at[j]], rows_v.at[p], gsem)

    def wback(j, p):
        return pltpu.make_async_copy(
            rows_v.at[p], out_hbm.at[j, pl.ds(b0, per_w)], wsem)

    depth = _NBUF - 1

    for k in range(depth):
        gather(k, k).start()

    gather(0, 0).wait()
    wback(0, 0).start()
    gather(depth, depth).start()

    def body(j, carry):
        p = lax.rem(j, _NBUF)
        pm1 = lax.rem(j + (_NBUF - 1), _NBUF)
        gather(j, p).wait()
        wback(j, p).start()
        wback(j - 1, pm1).wait()
        gather(j + depth, pm1).start()
        return carry

    lax.fori_loop(1, n_chunks - depth, body, 0)

    for j in range(n_chunks - depth, n_chunks):
        gather(j, j % _NBUF).wait()
        wback(j, j % _NBUF).start()
        wback(j - 1, (j - 1) % _NBUF).wait()
    wback(n_chunks - 1, (n_chunks - 1) % _NBUF).wait()


def kernel(tokens, table):
    b, s = tokens.shape
    v, d = table.shape
    assert b % _NUM_WORKERS == 0
    per_w = b // _NUM_WORKERS

    tok_t = tokens.astype(jnp.int32).T

    mesh = plsc.VectorSubcoreMesh(core_axis_name="c", subcore_axis_name="s")
    run = functools.partial(
        pl.kernel,
        mesh=mesh,
        out_type=jax.ShapeDtypeStruct((s, b, d), jnp.float32),
        scratch_types=[
            pltpu.VMEM((s, per_w), jnp.int32),
            pltpu.VMEM((_NBUF, per_w, d), jnp.float32),
            pltpu.SemaphoreType.DMA,
            pltpu.SemaphoreType.DMA,
        ],
    )(functools.partial(_emb_kernel, s, per_w))
    out3 = run(tok_t, table)
    return jnp.transpose(out3, (1, 0, 2))


# speedup vs baseline: 1.0150x; 1.0150x over previous
import functools

import jax
import jax.numpy as jnp
from jax import lax
from jax.experimental import pallas as pl
from jax.experimental.pallas import tpu as pltpu
from jax.experimental.pallas import tpu_sc as plsc

_NUM_WORKERS = 32
_NBUF = 4


def _emb_kernel(n_chunks, per_w,
                tok_hbm, table_hbm, out_hbm, idx_v, rows_v, gsem, wsem):
    wid = lax.axis_index("s") * 2 + lax.axis_index("c")
    b0 = wid * per_w
    pltpu.sync_copy(tok_hbm.at[:, pl.ds(b0, per_w)], idx_v)

    def gather(j, p):
        return pltpu.make_async_copy(
            table_hbm.at[idx_v.at[j]], rows_v.at[p], gsem)

    def wback(j, p):
        return pltpu.make_async_copy(
            rows_v.at[p], out_hbm.at[j, pl.ds(b0, per_w)], wsem)

    depth = _NBUF - 1

    for k in range(depth):
        gather(k, k).start()

    gather(0, 0).wait()
    wback(0, 0).start()
    gather(depth, depth).start()

    def body(j, carry):
        p = lax.rem(j, _NBUF)
        pm1 = lax.rem(j + (_NBUF - 1), _NBUF)
        gather(j, p).wait()
        wback(j, p).start()
        wback(j - 1, pm1).wait()
        gather(j + depth, pm1).start()
        return carry

    lax.fori_loop(1, n_chunks - depth, body, 0)

    for j in range(n_chunks - depth, n_chunks):
        gather(j, j % _NBUF).wait()
        wback(j, j % _NBUF).start()
        wback(j - 1, (j - 1) % _NBUF).wait()
    wback(n_chunks - 1, (n_chunks - 1) % _NBUF).wait()


def kernel(tokens, table):
    b, s = tokens.shape
    v, d = table.shape
    assert b % _NUM_WORKERS == 0
    per_w = b // _NUM_WORKERS

    tok_t = tokens.astype(jnp.int32).T

    mesh = plsc.VectorSubcoreMesh(core_axis_name="c", subcore_axis_name="s")
    run = functools.partial(
        pl.kernel,
        mesh=mesh,
        out_type=jax.ShapeDtypeStruct((s, b, d), jnp.float32),
        scratch_types=[
            pltpu.VMEM((s, per_w), jnp.int32),
            pltpu.VMEM((_NBUF, per_w, d), jnp.float32),
            pltpu.SemaphoreType.DMA,
            pltpu.SemaphoreType.DMA,
        ],
    )(functools.partial(_emb_kernel, s, per_w))
    out3 = run(tok_t, table)
    return jnp.transpose(out3, (1, 0, 2))
